# baseline (speedup 1.0000x reference)
import functools

import jax
import jax.numpy as jnp
from jax import lax
from jax.experimental import pallas as pl
from jax.experimental.pallas import tpu as pltpu
from jax.experimental.pallas import tpu_sc as plsc

N = 10000
D = 128
E = 320000
NC = 2
NS = 16
NW = NC * NS
EPW = E // NW
CHUNK = 50
NCHUNK = EPW // CHUNK
GRP = 8
ROWS_PER_SUB = N // NS
WB_ROWS = 624
ZCOPIES = ROWS_PER_SUB // CHUNK
DEG_PAD = 10240
DEG_PER_SUB = DEG_PAD // NS

_F32 = jnp.float32


def _make_sc_agg(with_deg: bool, nb: int):
  mesh = plsc.VectorSubcoreMesh(core_axis_name="c", subcore_axis_name="s")
  out_type = [jax.ShapeDtypeStruct((NC, N, D), _F32)]
  if with_deg:
    out_type.append(jax.ShapeDtypeStruct((NC, DEG_PAD), _F32))
  scratch = [
      pltpu.VMEM((2 * GRP, CHUNK), jnp.int32),
      pltpu.VMEM((2 * GRP, CHUNK), jnp.int32),
      [pltpu.VMEM((CHUNK, D), _F32) for _ in range(nb)],
  ]
  if with_deg:
    scratch += [
        pltpu.VMEM((128,), _F32),
        pltpu.VMEM((DEG_PER_SUB,), _F32),
    ]
  scratch += [pltpu.VMEM_SHARED((N, D), _F32)]
  if with_deg:
    scratch += [pltpu.VMEM_SHARED((DEG_PAD,), _F32)]
  scratch += [
      [pltpu.SemaphoreType.DMA for _ in range(nb)],
      pltpu.SemaphoreType.DMA,
  ]

  def body(h_hbm, src_hbm, dst_hbm, *rest):
    if with_deg:
      agg_out, deg_out = rest[0], rest[1]
      rest = rest[2:]
    else:
      agg_out = rest[0]
      rest = rest[1:]
    if with_deg:
      (idx_s, idx_d, rows, ones_v, zeros_d, agg_sh, deg_sh,
       sems, semi) = rest
    else:
      (idx_s, idx_d, rows, agg_sh, sems, semi) = rest
      ones_v = zeros_d = deg_sh = None
    rows0 = rows[0]

    cid = lax.axis_index("c")
    sid = lax.axis_index("s")
    wid = cid * NS + sid
    zv = jnp.zeros((16,), _F32)
    ov = jnp.full((16,), 1.0, _F32)

    def zrow(i, carry):
      for j in range(D // 16):
        rows0[i, pl.ds(j * 16, 16)] = zv
      return carry
    lax.fori_loop(0, CHUNK, zrow, 0)
    if with_deg:
      for j in range(128 // 16):
        ones_v[pl.ds(j * 16, 16)] = ov
      for j in range(DEG_PER_SUB // 16):
        zeros_d[pl.ds(j * 16, 16)] = zv

    for k in range(ZCOPIES):
      pltpu.sync_copy(
          rows0, agg_sh.at[pl.ds(sid * ROWS_PER_SUB + k * CHUNK, CHUNK)])
    _zrem = ROWS_PER_SUB - ZCOPIES * CHUNK
    if _zrem:
      pltpu.sync_copy(
          rows0.at[pl.ds(0, _zrem)],
          agg_sh.at[pl.ds(sid * ROWS_PER_SUB + ZCOPIES * CHUNK, _zrem)])
    if with_deg:
      pltpu.sync_copy(zeros_d, deg_sh.at[pl.ds(sid * DEG_PER_SUB,
                                               DEG_PER_SUB)])
    plsc.subcore_barrier()

    NGRP = NCHUNK // GRP

    def idx_fetch(g, half):
      base = pl.multiple_of(wid * NCHUNK + g * GRP, 8)
      dsts = idx_s.at[pl.ds(half * GRP, GRP)]
      dstd = idx_d.at[pl.ds(half * GRP, GRP)]
      return (
          pltpu.async_copy(src_hbm.at[pl.ds(base, GRP)], dsts, semi),
          pltpu.async_copy(dst_hbm.at[pl.ds(base, GRP)], dstd, semi),
      )

    def idx_wait(g, half):
      base = pl.multiple_of(wid * NCHUNK + g * GRP, 8)
      dsts = idx_s.at[pl.ds(half * GRP, GRP)]
      dstd = idx_d.at[pl.ds(half * GRP, GRP)]
      pltpu.make_async_copy(src_hbm.at[pl.ds(base, GRP)], dsts, semi).wait()
      pltpu.make_async_copy(dst_hbm.at[pl.ds(base, GRP)], dstd, semi).wait()

    idx_fetch(0, 0)

    def group(g, carry):
      half = lax.rem(g, 2)
      idx_wait(g, half)

      @pl.when(g < NGRP - 1)
      def _():
        idx_fetch(g + 1, 1 - half)

      off = half * GRP
      for r in range(nb - 1):
        pltpu.async_copy(h_hbm.at[idx_s.at[off + r]], rows[r % nb],
                         sems[r % nb])
      for r in range(GRP):
        if r + nb - 1 < GRP:
          pltpu.async_copy(h_hbm.at[idx_s.at[off + r + nb - 1]],
                           rows[(r + nb - 1) % nb], sems[(r + nb - 1) % nb])
        pltpu.make_async_copy(h_hbm.at[idx_s.at[off + r]], rows[r % nb],
                              sems[r % nb]).wait()
        pltpu.sync_copy(rows[r % nb], agg_sh.at[idx_d.at[off + r]],
                        add=True)
        if with_deg:
          pltpu.sync_copy(ones_v.at[pl.ds(0, CHUNK)],
                          deg_sh.at[idx_d.at[off + r]], add=True)
      return carry

    lax.fori_loop(0, NGRP, group, 0)
    plsc.subcore_barrier()

    wb_base = pl.multiple_of(sid * WB_ROWS, 8)
    pltpu.sync_copy(agg_sh.at[pl.ds(wb_base, WB_ROWS)],
                    agg_out.at[cid, pl.ds(wb_base, WB_ROWS)])

    @pl.when(sid == NS - 1)
    def _():
      pltpu.sync_copy(agg_sh.at[pl.ds(NS * WB_ROWS, N - NS * WB_ROWS)],
                      agg_out.at[cid, pl.ds(NS * WB_ROWS, N - NS * WB_ROWS)])
    if with_deg:
      @pl.when(sid == 0)
      def _():
        pltpu.sync_copy(deg_sh, deg_out.at[cid])

  return pl.kernel(body, out_type=out_type, mesh=mesh,
                   scratch_types=scratch)


_sc_agg_deg = _make_sc_agg(True, 4)
_sc_agg = _make_sc_agg(False, 4)


def _make_tc_layer(apply_tanh: bool):
  BLK = 1000

  def body(h_ref, a_ref, d_ref, ws_ref, wn_ref, b_ref, o_ref):
    a = a_ref[0] + a_ref[1]
    d = d_ref[0] + d_ref[1]
    hn = a / jnp.maximum(d, 1.0)
    out = jnp.dot(h_ref[...], ws_ref[...], preferred_element_type=_F32)
    out = out + jnp.dot(hn, wn_ref[...], preferred_element_type=_F32)
    out = out + b_ref[...]
    if apply_tanh:
      out = jnp.tanh(out)
    o_ref[...] = out

  return pl.pallas_call(
      body,
      grid=(N // BLK,),
      in_specs=[
          pl.BlockSpec((BLK, D), lambda i: (i, 0)),
          pl.BlockSpec((NC, BLK, D), lambda i: (0, i, 0)),
          pl.BlockSpec((NC, BLK, 1), lambda i: (0, i, 0)),
          pl.BlockSpec((D, D), lambda i: (0, 0)),
          pl.BlockSpec((D, D), lambda i: (0, 0)),
          pl.BlockSpec((1, D), lambda i: (0, 0)),
      ],
      out_specs=pl.BlockSpec((BLK, D), lambda i: (i, 0)),
      out_shape=jax.ShapeDtypeStruct((N, D), _F32),
  )


_tc_layer1 = _make_tc_layer(False)
_tc_layer2 = _make_tc_layer(True)


@jax.jit
def kernel(x, edge_index, W_self1, W_neigh1, b1, W_self2, W_neigh2, b2):
  src2 = edge_index[0].reshape(NW * NCHUNK, CHUNK)
  dst2 = edge_index[1].reshape(NW * NCHUNK, CHUNK)
  agg1, degp = _sc_agg_deg(x, src2, dst2)
  deg3 = degp[:, :N, None]
  h1 = _tc_layer1(x, agg1, deg3, W_self1, W_neigh1, b1.reshape(1, D))
  (agg2,) = _sc_agg(h1, src2, dst2)
  out = _tc_layer2(h1, agg2, deg3, W_self2, W_neigh2, b2.reshape(1, D))
  return out


# speedup vs baseline: 1.0881x; 1.0881x over previous
import functools

import jax
import jax.numpy as jnp
from jax import lax
from jax.experimental import pallas as pl
from jax.experimental.pallas import tpu as pltpu
from jax.experimental.pallas import tpu_sc as plsc

N = 10000
D = 128
E = 320000
NC = 2
NS = 16
NW = NC * NS
EPW = E // NW
CHUNK = 125
NCHUNK = EPW // CHUNK
GRP = 8
ROWS_PER_SUB = N // NS
WB_ROWS = 624
ZCOPIES = ROWS_PER_SUB // CHUNK
DEG_PAD = 10240
DEG_PER_SUB = DEG_PAD // NS

_F32 = jnp.float32


def _make_sc_agg(with_deg: bool, nb: int):
  mesh = plsc.VectorSubcoreMesh(core_axis_name="c", subcore_axis_name="s")
  out_type = [jax.ShapeDtypeStruct((NC, N, D), _F32)]
  if with_deg:
    out_type.append(jax.ShapeDtypeStruct((NC, DEG_PAD), _F32))
  scratch = [
      pltpu.VMEM((2 * GRP, CHUNK), jnp.int32),
      pltpu.VMEM((2 * GRP, CHUNK), jnp.int32),
      [pltpu.VMEM((CHUNK, D), _F32) for _ in range(nb)],
  ]
  if with_deg:
    scratch += [
        pltpu.VMEM((128,), _F32),
        pltpu.VMEM((DEG_PER_SUB,), _F32),
    ]
  scratch += [pltpu.VMEM_SHARED((N, D), _F32)]
  if with_deg:
    scratch += [pltpu.VMEM_SHARED((DEG_PAD,), _F32)]
  scratch += [
      [pltpu.SemaphoreType.DMA for _ in range(nb)],
      pltpu.SemaphoreType.DMA,
  ]

  def body(h_hbm, src_hbm, dst_hbm, *rest):
    if with_deg:
      agg_out, deg_out = rest[0], rest[1]
      rest = rest[2:]
    else:
      agg_out = rest[0]
      rest = rest[1:]
    if with_deg:
      (idx_s, idx_d, rows, ones_v, zeros_d, agg_sh, deg_sh,
       sems, semi) = rest
    else:
      (idx_s, idx_d, rows, agg_sh, sems, semi) = rest
      ones_v = zeros_d = deg_sh = None
    rows0 = rows[0]

    cid = lax.axis_index("c")
    sid = lax.axis_index("s")
    wid = cid * NS + sid
    zv = jnp.zeros((16,), _F32)
    ov = jnp.full((16,), 1.0, _F32)

    def zrow(i, carry):
      for j in range(D // 16):
        rows0[i, pl.ds(j * 16, 16)] = zv
      return carry
    lax.fori_loop(0, CHUNK, zrow, 0)
    if with_deg:
      for j in range(128 // 16):
        ones_v[pl.ds(j * 16, 16)] = ov
      for j in range(DEG_PER_SUB // 16):
        zeros_d[pl.ds(j * 16, 16)] = zv

    for k in range(ZCOPIES):
      pltpu.sync_copy(
          rows0, agg_sh.at[pl.ds(sid * ROWS_PER_SUB + k * CHUNK, CHUNK)])
    _zrem = ROWS_PER_SUB - ZCOPIES * CHUNK
    if _zrem:
      pltpu.sync_copy(
          rows0.at[pl.ds(0, _zrem)],
          agg_sh.at[pl.ds(sid * ROWS_PER_SUB + ZCOPIES * CHUNK, _zrem)])
    if with_deg:
      pltpu.sync_copy(zeros_d, deg_sh.at[pl.ds(sid * DEG_PER_SUB,
                                               DEG_PER_SUB)])
    plsc.subcore_barrier()

    NGRP = NCHUNK // GRP

    def idx_fetch(g, half):
      base = pl.multiple_of(wid * NCHUNK + g * GRP, 8)
      dsts = idx_s.at[pl.ds(half * GRP, GRP)]
      dstd = idx_d.at[pl.ds(half * GRP, GRP)]
      return (
          pltpu.async_copy(src_hbm.at[pl.ds(base, GRP)], dsts, semi),
          pltpu.async_copy(dst_hbm.at[pl.ds(base, GRP)], dstd, semi),
      )

    def idx_wait(g, half):
      base = pl.multiple_of(wid * NCHUNK + g * GRP, 8)
      dsts = idx_s.at[pl.ds(half * GRP, GRP)]
      dstd = idx_d.at[pl.ds(half * GRP, GRP)]
      pltpu.make_async_copy(src_hbm.at[pl.ds(base, GRP)], dsts, semi).wait()
      pltpu.make_async_copy(dst_hbm.at[pl.ds(base, GRP)], dstd, semi).wait()

    idx_fetch(0, 0)

    def group(g, carry):
      half = lax.rem(g, 2)
      idx_wait(g, half)

      @pl.when(g < NGRP - 1)
      def _():
        idx_fetch(g + 1, 1 - half)

      off = half * GRP
      for r in range(nb - 1):
        pltpu.async_copy(h_hbm.at[idx_s.at[off + r]], rows[r % nb],
                         sems[r % nb])
      for r in range(GRP):
        if r + nb - 1 < GRP:
          pltpu.async_copy(h_hbm.at[idx_s.at[off + r + nb - 1]],
                           rows[(r + nb - 1) % nb], sems[(r + nb - 1) % nb])
        pltpu.make_async_copy(h_hbm.at[idx_s.at[off + r]], rows[r % nb],
                              sems[r % nb]).wait()
        pltpu.sync_copy(rows[r % nb], agg_sh.at[idx_d.at[off + r]],
                        add=True)
        if with_deg:
          pltpu.sync_copy(ones_v.at[pl.ds(0, CHUNK)],
                          deg_sh.at[idx_d.at[off + r]], add=True)
      return carry

    lax.fori_loop(0, NGRP, group, 0)
    plsc.subcore_barrier()

    wb_base = pl.multiple_of(sid * WB_ROWS, 8)
    pltpu.sync_copy(agg_sh.at[pl.ds(wb_base, WB_ROWS)],
                    agg_out.at[cid, pl.ds(wb_base, WB_ROWS)])

    @pl.when(sid == NS - 1)
    def _():
      pltpu.sync_copy(agg_sh.at[pl.ds(NS * WB_ROWS, N - NS * WB_ROWS)],
                      agg_out.at[cid, pl.ds(NS * WB_ROWS, N - NS * WB_ROWS)])
    if with_deg:
      @pl.when(sid == 0)
      def _():
        pltpu.sync_copy(deg_sh, deg_out.at[cid])

  return pl.kernel(body, out_type=out_type, mesh=mesh,
                   scratch_types=scratch)


_sc_agg_deg = _make_sc_agg(True, 2)
_sc_agg = _make_sc_agg(False, 2)


BLK = 1000

_spec_rows = pl.BlockSpec((BLK, D), lambda i: (i, 0))
_spec_parts = pl.BlockSpec((NC, BLK, D), lambda i: (0, i, 0))
_spec_deg = pl.BlockSpec((NC, BLK, 1), lambda i: (0, i, 0))
_spec_w = pl.BlockSpec((D, D), lambda i: (0, 0))
_spec_b = pl.BlockSpec((1, D), lambda i: (0, 0))


def _tc_m_body(a_ref, d_ref, o_ref):
  d = d_ref[0] + d_ref[1]
  o_ref[...] = (a_ref[0] + a_ref[1]) / jnp.maximum(d, 1.0)


_tc_m = pl.pallas_call(
    _tc_m_body,
    grid=(N // BLK,),
    in_specs=[_spec_parts, _spec_deg],
    out_specs=_spec_rows,
    out_shape=jax.ShapeDtypeStruct((N, D), _F32),
)


def _tc_l1_body(h_ref, m_ref, ws_ref, wn_ref, b_ref, o_ref):
  out = jnp.dot(h_ref[...], ws_ref[...], preferred_element_type=_F32)
  out = out + jnp.dot(m_ref[...], wn_ref[...], preferred_element_type=_F32)
  o_ref[...] = out + b_ref[...]


_tc_l1 = pl.pallas_call(
    _tc_l1_body,
    grid=(N // BLK,),
    in_specs=[_spec_rows, _spec_rows, _spec_w, _spec_w, _spec_b],
    out_specs=_spec_rows,
    out_shape=jax.ShapeDtypeStruct((N, D), _F32),
)


def _tc_l2_body(h1_ref, p_ref, a1_ref, d_ref, ws1_ref, wn1_ref, b1_ref,
                ws2_ref, wn2_ref, b2_ref, o_ref):
  a1 = a1_ref[0] + a1_ref[1]
  pm = p_ref[0] + p_ref[1]
  d = d_ref[0] + d_ref[1]
  agg2 = jnp.dot(a1, ws1_ref[...], preferred_element_type=_F32)
  agg2 = agg2 + jnp.dot(pm, wn1_ref[...], preferred_element_type=_F32)
  agg2 = agg2 + d * b1_ref[...]
  hn2 = agg2 / jnp.maximum(d, 1.0)
  out = jnp.dot(h1_ref[...], ws2_ref[...], preferred_element_type=_F32)
  out = out + jnp.dot(hn2, wn2_ref[...], preferred_element_type=_F32)
  o_ref[...] = jnp.tanh(out + b2_ref[...])


_tc_l2 = pl.pallas_call(
    _tc_l2_body,
    grid=(N // BLK,),
    in_specs=[_spec_rows, _spec_parts, _spec_parts, _spec_deg,
              _spec_w, _spec_w, _spec_b, _spec_w, _spec_w, _spec_b],
    out_specs=_spec_rows,
    out_shape=jax.ShapeDtypeStruct((N, D), _F32),
)


@jax.jit
def kernel(x, edge_index, W_self1, W_neigh1, b1, W_self2, W_neigh2, b2):
  src2 = edge_index[0].reshape(NW * NCHUNK, CHUNK)
  dst2 = edge_index[1].reshape(NW * NCHUNK, CHUNK)
  agg1, degp = _sc_agg_deg(x, src2, dst2)
  deg3 = degp[:, :N, None]
  m = _tc_m(agg1, deg3)
  (p,) = _sc_agg(m, src2, dst2)
  h1 = _tc_l1(x, m, W_self1, W_neigh1, b1.reshape(1, D))
  out = _tc_l2(h1, p, agg1, deg3, W_self1, W_neigh1, b1.reshape(1, D),
               W_self2, W_neigh2, b2.reshape(1, D))
  return out


# speedup vs baseline: 1.1907x; 1.0943x over previous
import functools

import jax
import jax.numpy as jnp
from jax import lax
from jax.experimental import pallas as pl
from jax.experimental.pallas import tpu as pltpu
from jax.experimental.pallas import tpu_sc as plsc

N = 10000
D = 128
E = 320000
NC = 2
NS = 16
NW = NC * NS
EPW = E // NW
CHUNK = 125
NCHUNK = EPW // CHUNK
GRP = 8
ROWS_PER_SUB = N // NS
WB_ROWS = 624
ZCOPIES = ROWS_PER_SUB // CHUNK
DEG_PAD = 10240
DEG_PER_SUB = DEG_PAD // NS

_F32 = jnp.float32


def _make_sc_agg(with_deg: bool):
  mesh = plsc.VectorSubcoreMesh(core_axis_name="c", subcore_axis_name="s")
  out_type = [jax.ShapeDtypeStruct((NC, N, D), _F32)]
  if with_deg:
    out_type.append(jax.ShapeDtypeStruct((NC, DEG_PAD), _F32))
  scratch = [
      pltpu.VMEM((2 * GRP, CHUNK), jnp.int32),
      pltpu.VMEM((2 * GRP, CHUNK), jnp.int32),
      pltpu.VMEM((CHUNK, D), _F32),
      pltpu.VMEM((CHUNK, D), _F32),
      pltpu.VMEM((128,), _F32),
      pltpu.VMEM((DEG_PER_SUB,), _F32),
      pltpu.VMEM_SHARED((N, D), _F32),
      pltpu.VMEM_SHARED((DEG_PAD,), _F32),
      pltpu.SemaphoreType.DMA,
      pltpu.SemaphoreType.DMA,
      pltpu.SemaphoreType.DMA,
  ]

  def body(h_hbm, src_hbm, dst_hbm, *rest):
    if with_deg:
      agg_out, deg_out = rest[0], rest[1]
      rest = rest[2:]
    else:
      agg_out = rest[0]
      rest = rest[1:]
    (idx_s, idx_d, rows0, rows1, ones_v, zeros_d, agg_sh, deg_sh,
     sem0, sem1, semi) = rest
    rows = (rows0, rows1)
    sems = (sem0, sem1)

    cid = lax.axis_index("c")
    sid = lax.axis_index("s")
    wid = cid * NS + sid
    zv = jnp.zeros((16,), _F32)
    ov = jnp.full((16,), 1.0, _F32)

    def zrow(i, carry):
      for j in range(D // 16):
        rows0[i, pl.ds(j * 16, 16)] = zv
      return carry
    lax.fori_loop(0, CHUNK, zrow, 0)
    if with_deg:
      for j in range(128 // 16):
        ones_v[pl.ds(j * 16, 16)] = ov
      for j in range(DEG_PER_SUB // 16):
        zeros_d[pl.ds(j * 16, 16)] = zv

    for k in range(ZCOPIES):
      pltpu.sync_copy(
          rows0, agg_sh.at[pl.ds(sid * ROWS_PER_SUB + k * CHUNK, CHUNK)])
    if with_deg:
      pltpu.sync_copy(zeros_d, deg_sh.at[pl.ds(sid * DEG_PER_SUB,
                                               DEG_PER_SUB)])
    plsc.subcore_barrier()

    NGRP = NCHUNK // GRP

    def idx_fetch(g, half):
      base = pl.multiple_of(wid * NCHUNK + g * GRP, 8)
      dsts = idx_s.at[pl.ds(half * GRP, GRP)]
      dstd = idx_d.at[pl.ds(half * GRP, GRP)]
      return (
          pltpu.async_copy(src_hbm.at[pl.ds(base, GRP)], dsts, semi),
          pltpu.async_copy(dst_hbm.at[pl.ds(base, GRP)], dstd, semi),
      )

    def idx_wait(g, half):
      base = pl.multiple_of(wid * NCHUNK + g * GRP, 8)
      dsts = idx_s.at[pl.ds(half * GRP, GRP)]
      dstd = idx_d.at[pl.ds(half * GRP, GRP)]
      pltpu.make_async_copy(src_hbm.at[pl.ds(base, GRP)], dsts, semi).wait()
      pltpu.make_async_copy(dst_hbm.at[pl.ds(base, GRP)], dstd, semi).wait()

    def cpos(c):
      return lax.rem(c // GRP, 2) * GRP + lax.rem(c, GRP)

    idx_fetch(0, 0)
    idx_wait(0, 0)
    idx_fetch(1, 1)
    pltpu.async_copy(h_hbm.at[idx_s.at[0]], rows[0], sems[0])
    pltpu.async_copy(h_hbm.at[idx_s.at[1]], rows[1], sems[1])

    def step(cc, carry):
      c0 = cc * 2
      for j in range(2):
        c = c0 + j
        rb, sb = rows[j], sems[j]
        pltpu.make_async_copy(h_hbm.at[idx_s.at[cpos(c)]], rb, sb).wait()
        pltpu.sync_copy(rb, agg_sh.at[idx_d.at[cpos(c)]], add=True)
        if with_deg:
          pltpu.sync_copy(ones_v.at[pl.ds(0, CHUNK)],
                          deg_sh.at[idx_d.at[cpos(c)]], add=True)
        nxt = c + 2

        @pl.when((lax.rem(nxt, GRP) == 0) & (nxt < NCHUNK))
        def _():
          g = nxt // GRP
          idx_wait(g, lax.rem(g, 2))

        @pl.when((lax.rem(nxt, GRP) == 1) & (nxt < NCHUNK))
        def _():
          g2 = nxt // GRP + 1

          @pl.when(g2 < NGRP)
          def _():
            idx_fetch(g2, lax.rem(g2, 2))

        @pl.when(nxt < NCHUNK)
        def _():
          pltpu.async_copy(h_hbm.at[idx_s.at[cpos(nxt)]], rb, sb)
      return carry

    lax.fori_loop(0, NCHUNK // 2, step, 0)
    plsc.subcore_barrier()

    wb_base = pl.multiple_of(sid * WB_ROWS, 8)
    pltpu.sync_copy(agg_sh.at[pl.ds(wb_base, WB_ROWS)],
                    agg_out.at[cid, pl.ds(wb_base, WB_ROWS)])

    @pl.when(sid == NS - 1)
    def _():
      pltpu.sync_copy(agg_sh.at[pl.ds(NS * WB_ROWS, N - NS * WB_ROWS)],
                      agg_out.at[cid, pl.ds(NS * WB_ROWS, N - NS * WB_ROWS)])
    if with_deg:
      @pl.when(sid == 0)
      def _():
        pltpu.sync_copy(deg_sh, deg_out.at[cid])

  return pl.kernel(body, out_type=out_type, mesh=mesh,
                   scratch_types=scratch)


_sc_agg_deg = _make_sc_agg(True)
_sc_agg = _make_sc_agg(False)


def _make_tc_layer(apply_tanh: bool):
  BLK = 1000

  def body(h_ref, a_ref, d_ref, ws_ref, wn_ref, b_ref, o_ref):
    a = a_ref[0] + a_ref[1]
    d = d_ref[0] + d_ref[1]
    hn = a / jnp.maximum(d, 1.0)
    out = jnp.dot(h_ref[...], ws_ref[...], preferred_element_type=_F32)
    out = out + jnp.dot(hn, wn_ref[...], preferred_element_type=_F32)
    out = out + b_ref[...]
    if apply_tanh:
      out = jnp.tanh(out)
    o_ref[...] = out

  return pl.pallas_call(
      body,
      grid=(N // BLK,),
      in_specs=[
          pl.BlockSpec((BLK, D), lambda i: (i, 0)),
          pl.BlockSpec((NC, BLK, D), lambda i: (0, i, 0)),
          pl.BlockSpec((NC, BLK, 1), lambda i: (0, i, 0)),
          pl.BlockSpec((D, D), lambda i: (0, 0)),
          pl.BlockSpec((D, D), lambda i: (0, 0)),
          pl.BlockSpec((1, D), lambda i: (0, 0)),
      ],
      out_specs=pl.BlockSpec((BLK, D), lambda i: (i, 0)),
      out_shape=jax.ShapeDtypeStruct((N, D), _F32),
  )


_tc_layer1 = _make_tc_layer(False)
_tc_layer2 = _make_tc_layer(True)


@jax.jit
def kernel(x, edge_index, W_self1, W_neigh1, b1, W_self2, W_neigh2, b2):
  src2 = edge_index[0].reshape(NW * NCHUNK, CHUNK)
  dst2 = edge_index[1].reshape(NW * NCHUNK, CHUNK)
  agg1, degp = _sc_agg_deg(x, src2, dst2)
  deg3 = degp[:, :N, None]
  h1 = _tc_layer1(x, agg1, deg3, W_self1, W_neigh1, b1.reshape(1, D))
  (agg2,) = _sc_agg(h1, src2, dst2)
  out = _tc_layer2(h1, agg2, deg3, W_self2, W_neigh2, b2.reshape(1, D))
  return out


# speedup vs baseline: 1.2092x; 1.0155x over previous
import functools

import jax
import jax.numpy as jnp
from jax import lax
from jax.experimental import pallas as pl
from jax.experimental.pallas import tpu as pltpu
from jax.experimental.pallas import tpu_sc as plsc

N = 10000
D = 128
E = 320000
NC = 2
NS = 16
NW = NC * NS
EPW = E // NW
CHUNK = 125
NCHUNK = EPW // CHUNK
GRP = 8
ROWS_PER_SUB = N // NS
WB_ROWS = 624
ZCOPIES = ROWS_PER_SUB // CHUNK
DEG_PAD = 10240
DEG_PER_SUB = DEG_PAD // NS

_F32 = jnp.float32


def _make_sc_agg(with_deg: bool):
  mesh = plsc.VectorSubcoreMesh(core_axis_name="c", subcore_axis_name="s")
  out_type = [jax.ShapeDtypeStruct((NC, N, D), _F32)]
  if with_deg:
    out_type.append(jax.ShapeDtypeStruct((NC, DEG_PAD), _F32))
  scratch = [
      pltpu.VMEM((2 * GRP, CHUNK), jnp.int32),
      pltpu.VMEM((2 * GRP, CHUNK), jnp.int32),
      pltpu.VMEM((CHUNK, D), _F32),
      pltpu.VMEM((CHUNK, D), _F32),
      pltpu.VMEM((128,), _F32),
      pltpu.VMEM((DEG_PER_SUB,), _F32),
      pltpu.VMEM_SHARED((N, D), _F32),
      pltpu.VMEM_SHARED((DEG_PAD,), _F32),
      pltpu.SemaphoreType.DMA,
      pltpu.SemaphoreType.DMA,
      pltpu.SemaphoreType.DMA,
  ]

  def body(h_hbm, src_hbm, dst_hbm, *rest):
    if with_deg:
      agg_out, deg_out = rest[0], rest[1]
      rest = rest[2:]
    else:
      agg_out = rest[0]
      rest = rest[1:]
    (idx_s, idx_d, rows0, rows1, ones_v, zeros_d, agg_sh, deg_sh,
     sem0, sem1, semi) = rest
    rows = (rows0, rows1)
    sems = (sem0, sem1)

    cid = lax.axis_index("c")
    sid = lax.axis_index("s")
    wid = cid * NS + sid
    zv = jnp.zeros((16,), _F32)
    ov = jnp.full((16,), 1.0, _F32)

    def zrow(i, carry):
      for j in range(D // 16):
        rows0[i, pl.ds(j * 16, 16)] = zv
      return carry
    lax.fori_loop(0, CHUNK, zrow, 0)
    if with_deg:
      for j in range(128 // 16):
        ones_v[pl.ds(j * 16, 16)] = ov
      for j in range(DEG_PER_SUB // 16):
        zeros_d[pl.ds(j * 16, 16)] = zv

    for k in range(ZCOPIES):
      pltpu.sync_copy(
          rows0, agg_sh.at[pl.ds(sid * ROWS_PER_SUB + k * CHUNK, CHUNK)])
    if with_deg:
      pltpu.sync_copy(zeros_d, deg_sh.at[pl.ds(sid * DEG_PER_SUB,
                                               DEG_PER_SUB)])
    plsc.subcore_barrier()

    NGRP = NCHUNK // GRP

    def idx_fetch(g, half):
      base = pl.multiple_of(wid * NCHUNK + g * GRP, 8)
      dsts = idx_s.at[pl.ds(half * GRP, GRP)]
      dstd = idx_d.at[pl.ds(half * GRP, GRP)]
      return (
          pltpu.async_copy(src_hbm.at[pl.ds(base, GRP)], dsts, semi),
          pltpu.async_copy(dst_hbm.at[pl.ds(base, GRP)], dstd, semi),
      )

    def idx_wait(g, half):
      base = pl.multiple_of(wid * NCHUNK + g * GRP, 8)
      dsts = idx_s.at[pl.ds(half * GRP, GRP)]
      dstd = idx_d.at[pl.ds(half * GRP, GRP)]
      pltpu.make_async_copy(src_hbm.at[pl.ds(base, GRP)], dsts, semi).wait()
      pltpu.make_async_copy(dst_hbm.at[pl.ds(base, GRP)], dstd, semi).wait()

    def cpos(c):
      return lax.rem(c // GRP, 2) * GRP + lax.rem(c, GRP)

    idx_fetch(0, 0)
    idx_wait(0, 0)
    idx_fetch(1, 1)
    pltpu.async_copy(h_hbm.at[idx_s.at[0]], rows[0], sems[0])
    pltpu.async_copy(h_hbm.at[idx_s.at[1]], rows[1], sems[1])

    def step(cc, carry):
      c0 = cc * 2
      for j in range(2):
        c = c0 + j
        rb, sb = rows[j], sems[j]
        pltpu.make_async_copy(h_hbm.at[idx_s.at[cpos(c)]], rb, sb).wait()
        pltpu.sync_copy(rb, agg_sh.at[idx_d.at[cpos(c)]], add=True)
        if with_deg:
          pltpu.sync_copy(ones_v.at[pl.ds(0, CHUNK)],
                          deg_sh.at[idx_d.at[cpos(c)]], add=True)
        nxt = c + 2

        @pl.when((lax.rem(nxt, GRP) == 0) & (nxt < NCHUNK))
        def _():
          g = nxt // GRP
          idx_wait(g, lax.rem(g, 2))

        @pl.when((lax.rem(nxt, GRP) == 1) & (nxt < NCHUNK))
        def _():
          g2 = nxt // GRP + 1

          @pl.when(g2 < NGRP)
          def _():
            idx_fetch(g2, lax.rem(g2, 2))

        @pl.when(nxt < NCHUNK)
        def _():
          pltpu.async_copy(h_hbm.at[idx_s.at[cpos(nxt)]], rb, sb)
      return carry

    lax.fori_loop(0, NCHUNK // 2, step, 0)
    plsc.subcore_barrier()

    wb_base = pl.multiple_of(sid * WB_ROWS, 8)
    pltpu.sync_copy(agg_sh.at[pl.ds(wb_base, WB_ROWS)],
                    agg_out.at[cid, pl.ds(wb_base, WB_ROWS)])

    @pl.when(sid == NS - 1)
    def _():
      pltpu.sync_copy(agg_sh.at[pl.ds(NS * WB_ROWS, N - NS * WB_ROWS)],
                      agg_out.at[cid, pl.ds(NS * WB_ROWS, N - NS * WB_ROWS)])
    if with_deg:
      @pl.when(sid == 0)
      def _():
        pltpu.sync_copy(deg_sh, deg_out.at[cid])

  return pl.kernel(body, out_type=out_type, mesh=mesh,
                   scratch_types=scratch)


_sc_agg_deg = _make_sc_agg(True)
_sc_agg = _make_sc_agg(False)


def _make_tc_layer(apply_tanh: bool):
  BLK = 2000

  def body(h_ref, a_ref, d_ref, ws_ref, wn_ref, b_ref, o_ref):
    a = a_ref[0] + a_ref[1]
    d = d_ref[0] + d_ref[1]
    hn = a / jnp.maximum(d, 1.0)
    out = jnp.dot(h_ref[...], ws_ref[...], preferred_element_type=_F32)
    out = out + jnp.dot(hn, wn_ref[...], preferred_element_type=_F32)
    out = out + b_ref[...]
    if apply_tanh:
      out = jnp.tanh(out)
    o_ref[...] = out

  return pl.pallas_call(
      body,
      grid=(N // BLK,),
      in_specs=[
          pl.BlockSpec((BLK, D), lambda i: (i, 0)),
          pl.BlockSpec((NC, BLK, D), lambda i: (0, i, 0)),
          pl.BlockSpec((NC, BLK, 1), lambda i: (0, i, 0)),
          pl.BlockSpec((D, D), lambda i: (0, 0)),
          pl.BlockSpec((D, D), lambda i: (0, 0)),
          pl.BlockSpec((1, D), lambda i: (0, 0)),
      ],
      out_specs=pl.BlockSpec((BLK, D), lambda i: (i, 0)),
      out_shape=jax.ShapeDtypeStruct((N, D), _F32),
  )


_tc_layer1 = _make_tc_layer(False)
_tc_layer2 = _make_tc_layer(True)


@jax.jit
def kernel(x, edge_index, W_self1, W_neigh1, b1, W_self2, W_neigh2, b2):
  src2 = edge_index[0].reshape(NW * NCHUNK, CHUNK)
  dst2 = edge_index[1].reshape(NW * NCHUNK, CHUNK)
  agg1, degp = _sc_agg_deg(x, src2, dst2)
  deg3 = degp[:, :N, None]
  h1 = _tc_layer1(x, agg1, deg3, W_self1, W_neigh1, b1.reshape(1, D))
  (agg2,) = _sc_agg(h1, src2, dst2)
  out = _tc_layer2(h1, agg2, deg3, W_self2, W_neigh2, b2.reshape(1, D))
  return out
